# baseline (speedup 1.0000x reference)
import functools

import jax
import jax.numpy as jnp
from jax import lax
from jax.experimental import pallas as pl
from jax.experimental.pallas import tpu as pltpu
from jax.experimental.pallas import tpu_sc as plsc

NUM_FIELDS = 26
VOCAB_P1 = 100001
EMB_DIM = 32
BATCH = 16384

NC, NS, L = 2, 16, 16
NW = NC * NS
ROWS = BATCH * NUM_FIELDS
PER_W = ROWS // NW
CHUNK = 128
NCH = PER_W // CHUNK
NBUF = 4
GROUPS = NCH // NBUF

_MESH = plsc.VectorSubcoreMesh(
    core_axis_name="c", subcore_axis_name="s", num_cores=NC, num_subcores=NS
)


@functools.partial(
    pl.kernel,
    out_type=jax.ShapeDtypeStruct((ROWS, EMB_DIM), jnp.float32),
    mesh=_MESH,
    scratch_types=[
        pltpu.VMEM((PER_W,), jnp.int32),
        pltpu.VMEM((NCH, CHUNK), jnp.int32),
        pltpu.VMEM((NBUF, CHUNK, EMB_DIM), jnp.float32),
        pltpu.SemaphoreType.DMA,
    ],
    compiler_params=pltpu.CompilerParams(use_tc_tiling_on_sc=False),
)
def _embed_kernel(x_hbm, tab_hbm, out_hbm, xloc, idxm, rows, sem):
    cid = lax.axis_index("c")
    sid = lax.axis_index("s")
    wid = sid * NC + cid
    base = wid * PER_W

    pltpu.sync_copy(x_hbm.at[pl.ds(base, PER_W)], xloc)

    lanes = lax.iota(jnp.int32, L)

    def idx_body(c, carry):
        for g in range(CHUNK // L):
            off = c * CHUNK + g * L
            pos = lanes + off
            field = lax.rem(pos, NUM_FIELDS)
            idxm[c, pl.ds(g * L, L)] = xloc[pl.ds(off, L)] + field * VOCAB_P1
        return carry

    lax.fori_loop(0, NCH, idx_body, 0)

    def gather_body(grp, carry):
        c0 = grp * NBUF
        descs = [
            pltpu.async_copy(tab_hbm.at[idxm.at[c0 + b]], rows.at[b], sem)
            for b in range(NBUF)
        ]
        for b in range(NBUF):
            descs[b].wait()
            pltpu.sync_copy(
                rows.at[b], out_hbm.at[pl.ds(base + (c0 + b) * CHUNK, CHUNK)]
            )
        return carry

    lax.fori_loop(0, GROUPS, gather_body, 0)


def kernel(x, tables):
    x_flat = x.reshape(ROWS).astype(jnp.int32)
    tab_flat = tables.reshape(NUM_FIELDS * VOCAB_P1, EMB_DIM)
    out = _embed_kernel(x_flat, tab_flat)
    return out.reshape(BATCH, NUM_FIELDS * EMB_DIM)


# speedup vs baseline: 2.4783x; 2.4783x over previous
import functools

import jax
import jax.numpy as jnp
from jax import lax
from jax.experimental import pallas as pl
from jax.experimental.pallas import tpu as pltpu
from jax.experimental.pallas import tpu_sc as plsc

NUM_FIELDS = 26
VOCAB_P1 = 100001
EMB_DIM = 32
BATCH = 16384

NC, NS, L = 2, 16, 16
NW = NC * NS
PER_B = BATCH // NW
CHUNK = 128
CPF = PER_B // CHUNK
NBUF = CPF

_MESH = plsc.VectorSubcoreMesh(
    core_axis_name="c", subcore_axis_name="s", num_cores=NC, num_subcores=NS
)


@functools.partial(
    pl.kernel,
    out_type=jax.ShapeDtypeStruct((BATCH, NUM_FIELDS * EMB_DIM), jnp.float32),
    mesh=_MESH,
    scratch_types=[
        pltpu.VMEM((PER_B, NUM_FIELDS), jnp.int32),
        pltpu.VMEM((PER_B,), jnp.int32),
        pltpu.VMEM((NBUF, CHUNK, EMB_DIM), jnp.float32),
        pltpu.SemaphoreType.DMA,
    ],
    compiler_params=pltpu.CompilerParams(
        use_tc_tiling_on_sc=False, needs_layout_passes=False
    ),
)
def _embed_kernel(x_hbm, tab_hbm, out_hbm, xloc, idxb, rows, sem):
    cid = lax.axis_index("c")
    sid = lax.axis_index("s")
    wid = sid * NC + cid
    b0 = wid * PER_B

    pltpu.sync_copy(x_hbm.at[pl.ds(b0, PER_B)], xloc)

    lanes = lax.iota(jnp.int32, L)

    def field_body(f, carry):
        fvec = jnp.full((L,), f, dtype=jnp.int32)

        def ext(j, c2):
            idxb[pl.ds(j * L, L)] = plsc.load_gather(
                xloc, [lanes + j * L, fvec]
            )
            return c2

        lax.fori_loop(0, PER_B // L, ext, 0)

        tab_f = tab_hbm.at[f]
        descs = [
            pltpu.async_copy(
                tab_f.at[idxb.at[pl.ds(b * CHUNK, CHUNK)]], rows.at[b], sem
            )
            for b in range(CPF)
        ]
        for b in range(CPF):
            descs[b].wait()
            pltpu.sync_copy(
                rows.at[b],
                out_hbm.at[
                    pl.ds(b0 + b * CHUNK, CHUNK),
                    pl.ds(f * EMB_DIM, EMB_DIM),
                ],
            )
        return carry

    lax.fori_loop(0, NUM_FIELDS, field_body, 0)


def kernel(x, tables):
    return _embed_kernel(x.astype(jnp.int32), tables)


# speedup vs baseline: 28.4959x; 11.4981x over previous
import functools

import jax
import jax.numpy as jnp
from jax import lax
from jax.experimental import pallas as pl
from jax.experimental.pallas import tpu as pltpu
from jax.experimental.pallas import tpu_sc as plsc

NUM_FIELDS = 26
VOCAB_P1 = 100001
EMB_DIM = 32
BATCH = 16384

NC, NS, L = 2, 16, 16
NW = NC * NS
JROWS = NUM_FIELDS * EMB_DIM
JPW = JROWS // NW
OCH = 4096
NOCH = BATCH // OCH

_MESH = plsc.VectorSubcoreMesh(
    core_axis_name="c", subcore_axis_name="s", num_cores=NC, num_subcores=NS
)


@functools.partial(
    pl.kernel,
    out_type=jax.ShapeDtypeStruct((JROWS, BATCH), jnp.float32),
    mesh=_MESH,
    scratch_types=[
        pltpu.VMEM((VOCAB_P1,), jnp.float32),
        pltpu.VMEM((BATCH,), jnp.int32),
        pltpu.VMEM((OCH,), jnp.float32),
    ],
    compiler_params=pltpu.CompilerParams(
        use_tc_tiling_on_sc=True, needs_layout_passes=False
    ),
)
def _embed_kernel(xt_hbm, tab_hbm, out_hbm, rowbuf, idxb, outb):
    cid = lax.axis_index("c")
    sid = lax.axis_index("s")
    wid = sid * NC + cid
    j0 = wid * JPW

    def pair_body(k, carry):
        j = j0 + k
        f = j // EMB_DIM
        c = j % EMB_DIM
        pltpu.sync_copy(xt_hbm.at[f], idxb)
        pltpu.sync_copy(tab_hbm.at[f, c], rowbuf)

        def chunk_body(q, c2):
            b0 = q * OCH

            def grp_body(g, c3):
                v = idxb[pl.ds(b0 + g * L, L)]
                outb[pl.ds(g * L, L)] = plsc.load_gather(rowbuf, [v])
                return c3

            lax.fori_loop(0, OCH // L, grp_body, 0)
            pltpu.sync_copy(outb, out_hbm.at[j, pl.ds(b0, OCH)])
            return c2

        lax.fori_loop(0, NOCH, chunk_body, 0)
        return carry

    lax.fori_loop(0, JPW, pair_body, 0)


def kernel(x, tables):
    xt = x.T.astype(jnp.int32)
    tabt = jnp.transpose(tables, (0, 2, 1))
    out_t = _embed_kernel(xt, tabt)
    return out_t.T


# speedup vs baseline: 31.9831x; 1.1224x over previous
import functools

import jax
import jax.numpy as jnp
from jax import lax
from jax.experimental import pallas as pl
from jax.experimental.pallas import tpu as pltpu
from jax.experimental.pallas import tpu_sc as plsc

NUM_FIELDS = 26
VOCAB_P1 = 100001
EMB_DIM = 32
BATCH = 16384

NC, NS, L = 2, 16, 16
NW = NC * NS
JROWS = NUM_FIELDS * EMB_DIM
JPW = JROWS // NW
OCH = 4096
NOCH = BATCH // OCH
UNROLL = 8

_MESH = plsc.VectorSubcoreMesh(
    core_axis_name="c", subcore_axis_name="s", num_cores=NC, num_subcores=NS
)


@functools.partial(
    pl.kernel,
    out_type=jax.ShapeDtypeStruct((JROWS, BATCH), jnp.float32),
    mesh=_MESH,
    scratch_types=[
        pltpu.VMEM((VOCAB_P1,), jnp.float32),
        pltpu.VMEM((BATCH,), jnp.int32),
        pltpu.VMEM((2, OCH), jnp.float32),
        pltpu.SemaphoreType.DMA,
    ],
    compiler_params=pltpu.CompilerParams(
        use_tc_tiling_on_sc=True, needs_layout_passes=False
    ),
)
def _embed_kernel(xt_hbm, tab_hbm, out_hbm, rowbuf, idxb, outb, sem):
    cid = lax.axis_index("c")
    sid = lax.axis_index("s")
    wid = sid * NC + cid
    j0 = wid * JPW

    def pair_body(k, fprev):
        j = j0 + k
        f = j // EMB_DIM
        c = j % EMB_DIM

        @pl.when(f != fprev)
        def _():
            pltpu.sync_copy(xt_hbm.at[f], idxb)

        pltpu.sync_copy(tab_hbm.at[f, c], rowbuf)

        descs = [None, None]
        for q in range(NOCH):
            b0 = q * OCH
            p = q % 2
            if descs[p] is not None:
                descs[p].wait()

            def grp_body(g, c3, b0=b0, p=p):
                base = b0 + g * (L * UNROLL)
                lbase = g * (L * UNROLL)
                for u in range(UNROLL):
                    v = idxb[pl.ds(base + u * L, L)]
                    outb[p, pl.ds(lbase + u * L, L)] = plsc.load_gather(
                        rowbuf, [v]
                    )
                return c3

            lax.fori_loop(0, OCH // (L * UNROLL), grp_body, 0)
            descs[p] = pltpu.async_copy(
                outb.at[p], out_hbm.at[j, pl.ds(b0, OCH)], sem
            )
        for d in descs:
            d.wait()
        return f

    lax.fori_loop(0, JPW, pair_body, jnp.int32(-1))


def kernel(x, tables):
    xt = x.T.astype(jnp.int32)
    tabt = jnp.transpose(tables, (0, 2, 1))
    out_t = _embed_kernel(xt, tabt)
    return out_t.T


# speedup vs baseline: 63.6315x; 1.9895x over previous
import functools

import jax
import jax.numpy as jnp
from jax import lax
from jax.experimental import pallas as pl
from jax.experimental.pallas import tpu as pltpu
from jax.experimental.pallas import tpu_sc as plsc

NUM_FIELDS = 26
VOCAB_P1 = 100001
EMB_DIM = 32
BATCH = 16384

NC, NS, L = 2, 16, 16
NW = NC * NS
JROWS = NUM_FIELDS * EMB_DIM
JPW = JROWS // NW
OCH = 4096
NOCH = BATCH // OCH
UNROLL = 8

_MESH = plsc.VectorSubcoreMesh(
    core_axis_name="c", subcore_axis_name="s", num_cores=NC, num_subcores=NS
)


@functools.partial(
    pl.kernel,
    out_type=jax.ShapeDtypeStruct((JROWS, BATCH), jnp.float32),
    mesh=_MESH,
    scratch_types=[
        pltpu.VMEM((VOCAB_P1,), jnp.float32),
        pltpu.VMEM((BATCH,), jnp.int32),
        pltpu.VMEM((2, OCH), jnp.float32),
        pltpu.SemaphoreType.DMA,
    ],
    compiler_params=pltpu.CompilerParams(
        use_tc_tiling_on_sc=True, needs_layout_passes=False
    ),
)
def _embed_kernel(xt_hbm, tab_hbm, out_hbm, rowbuf, idxb, outb, sem):
    cid = lax.axis_index("c")
    sid = lax.axis_index("s")
    wid = sid * NC + cid
    j0 = wid * JPW

    def pair_body(k, fprev):
        j = j0 + k
        f = j // EMB_DIM
        c = j % EMB_DIM

        @pl.when(f != fprev)
        def _():
            pltpu.sync_copy(xt_hbm.at[f], idxb)

        pltpu.sync_copy(tab_hbm.at[f, c], rowbuf)

        descs = [None, None]
        for q in range(NOCH):
            b0 = q * OCH
            p = q % 2
            if descs[p] is not None:
                descs[p].wait()

            @plsc.parallel_loop(0, OCH, step=L, unroll=UNROLL)
            def _gather(off, b0=b0, p=p):
                v = idxb[pl.ds(b0 + off, L)]
                outb[p, pl.ds(off, L)] = plsc.load_gather(rowbuf, [v])
            descs[p] = pltpu.async_copy(
                outb.at[p], out_hbm.at[j, pl.ds(b0, OCH)], sem
            )
        for d in descs:
            d.wait()
        return f

    lax.fori_loop(0, JPW, pair_body, jnp.int32(-1))


def kernel(x, tables):
    xt = x.T.astype(jnp.int32)
    tabt = jnp.transpose(tables, (0, 2, 1))
    out_t = _embed_kernel(xt, tabt)
    return out_t.T
